# Initial kernel scaffold; baseline (speedup 1.0000x reference)
import jax
import jax.numpy as jnp
from jax.experimental import pallas as pl


def kernel(x, spatial, W, b, gamma, beta):
    raise NotImplementedError("write your pallas kernel here")


# speedup vs baseline: 6.4238x; 6.4238x over previous
import functools

import jax
import jax.numpy as jnp
from jax import lax
from jax.experimental import pallas as pl
from jax.experimental.pallas import tpu as pltpu
from jax.experimental.pallas import tpu_sc as plsc

B = 4
N = 2048
HALF = N // 2
FEAT = 256
SDIM = 5
OUT = 512
K = 16

_NC = 2
_NS = 16
_NW = _NC * _NS
_ROWS_PER_W = (B * HALF * K) // _NW
_CHUNK = 128


def _fps_body(spt_ref, out_ref):
    spt = spt_ref[...]
    iota = lax.broadcasted_iota(jnp.int32, (B, N), 1)
    iota_h = lax.broadcasted_iota(jnp.int32, (B, HALF), 1)
    onehot0 = (iota == 0).astype(jnp.float32)
    rem0 = 1.0 - onehot0
    dist0 = jnp.full((B, N), jnp.inf, dtype=jnp.float32)
    sel0 = jnp.zeros((B, HALF), jnp.int32)

    def body(i, carry):
        dist, rem, onehot, sel = carry
        sq = []
        for d in range(SDIM):
            row = spt[:, d, :]
            cd = jnp.sum(onehot * row, axis=-1, keepdims=True)
            diff = cd - row
            sq.append(diff * diff)
        d2 = ((sq[0] + sq[4]) + sq[2]) + (sq[1] + sq[3])
        dist = jnp.where(rem > 0, jnp.minimum(d2, dist), dist)
        masked = jnp.where(rem > 0, dist, -jnp.inf)
        m = jnp.max(masked, axis=-1, keepdims=True)
        g = jnp.min(jnp.where(masked == m, iota, jnp.int32(1 << 30)),
                    axis=-1, keepdims=True)
        sel = jnp.where(iota_h == i, g, sel)
        rank_f = jnp.sum(jnp.where(iota < g, rem, 0.0), axis=-1, keepdims=True)
        rank = rank_f.astype(jnp.int32)
        rem = jnp.where(iota == rank, 0.0, rem)
        onehot = (iota == g).astype(jnp.float32)
        return dist, rem, onehot, sel

    _, _, _, sel = lax.fori_loop(1, HALF, body, (dist0, rem0, onehot0, sel0),
                                 unroll=False)
    out_ref[...] = sel


def _fps(sp_t):
    return pl.pallas_call(
        _fps_body,
        out_shape=jax.ShapeDtypeStruct((B, HALF), jnp.int32),
        in_specs=[pl.BlockSpec((B, SDIM, N), lambda: (0, 0, 0))],
        out_specs=pl.BlockSpec((B, HALF), lambda: (0, 0)),
    )(sp_t)


_CB = 256
_NCB = HALF // _CB


def _knn_body(spt_ref, idx_ref, keep_ref, cent_ref):
    spt = spt_ref[0]
    cidx = idx_ref[0]
    iota = lax.broadcasted_iota(jnp.int32, (_CB, N), 1)
    onehot = (iota == cidx).astype(jnp.float32)

    sq = []
    cols = []
    for d in range(SDIM):
        row = spt[d:d + 1, :]
        cd = jnp.sum(onehot * row, axis=-1, keepdims=True)
        cols.append(cd)
        diff = cd - row
        sq.append(diff * diff)
    cent_ref[0] = jnp.concatenate(cols, axis=-1)
    acc = ((sq[0] + sq[4]) + sq[2]) + (sq[1] + sq[3])
    dist = jnp.sqrt(acc)

    idxs = []
    for _ in range(K + 1):
        m = jnp.min(dist, axis=-1, keepdims=True)
        j = jnp.min(jnp.where(dist == m, iota, jnp.int32(1 << 30)),
                    axis=-1, keepdims=True)
        idxs.append(j)
        dist = jnp.where(iota == j, jnp.inf, dist)

    cum = jnp.zeros_like(cidx, dtype=jnp.bool_)
    keep = []
    for t in range(K):
        cum = jnp.logical_or(cum, idxs[t] == cidx)
        keep.append(jnp.where(cum, idxs[t + 1], idxs[t]))
    keep_ref[0] = jnp.concatenate(keep, axis=-1)


def _knn(sp_t, idx3):
    grid = (B, _NCB)
    return pl.pallas_call(
        _knn_body,
        grid=grid,
        out_shape=(
            jax.ShapeDtypeStruct((B, HALF, K), jnp.int32),
            jax.ShapeDtypeStruct((B, HALF, SDIM), jnp.float32),
        ),
        in_specs=[
            pl.BlockSpec((1, SDIM, N), lambda b, j: (b, 0, 0)),
            pl.BlockSpec((1, _CB, 1), lambda b, j: (b, j, 0)),
        ],
        out_specs=(
            pl.BlockSpec((1, _CB, K), lambda b, j: (b, j, 0)),
            pl.BlockSpec((1, _CB, SDIM), lambda b, j: (b, j, 0)),
        ),
    )(sp_t, idx3)


_LB = 256


def _linear_body(x_ref, sp_ref, wf_ref, ws_ref, b_ref, z_ref):
    z = jnp.dot(x_ref[0], wf_ref[...], preferred_element_type=jnp.float32)
    z = z + jnp.dot(sp_ref[0], ws_ref[...], preferred_element_type=jnp.float32)
    z_ref[0] = z + b_ref[...]


def _linear(x, sp, wf_t, ws_t, bias2):
    grid = (B, N // _LB)
    return pl.pallas_call(
        _linear_body,
        grid=grid,
        out_shape=jax.ShapeDtypeStruct((B, N, OUT), jnp.float32),
        in_specs=[
            pl.BlockSpec((1, _LB, FEAT), lambda b, j: (b, j, 0)),
            pl.BlockSpec((1, _LB, SDIM), lambda b, j: (b, j, 0)),
            pl.BlockSpec((FEAT, OUT), lambda b, j: (0, 0)),
            pl.BlockSpec((SDIM, OUT), lambda b, j: (0, 0)),
            pl.BlockSpec((1, OUT), lambda b, j: (0, 0)),
        ],
        out_specs=pl.BlockSpec((1, _LB, OUT), lambda b, j: (b, j, 0)),
    )(x, sp, wf_t, ws_t, bias2)


def _sc_gather_body(table_hbm, idx_hbm, out_hbm, idx_v, rows_v, sem):
    wid = lax.axis_index("s") * _NC + lax.axis_index("c")
    nchunks = _ROWS_PER_W // _CHUNK
    pltpu.sync_copy(idx_hbm.at[pl.ds(wid * nchunks, nchunks)], idx_v)

    def chunk(j, _):
        pltpu.async_copy(table_hbm.at[idx_v.at[j]], rows_v, sem).wait()
        pltpu.sync_copy(
            rows_v, out_hbm.at[pl.ds(wid * _ROWS_PER_W + j * _CHUNK, _CHUNK)])
        return 0

    lax.fori_loop(0, nchunks, chunk, 0, unroll=False)


def _sc_gather(table, idx2):
    mesh = plsc.VectorSubcoreMesh(
        core_axis_name="c", subcore_axis_name="s",
        num_cores=_NC, num_subcores=_NS)
    kern = pl.kernel(
        _sc_gather_body,
        out_type=jax.ShapeDtypeStruct((B * HALF * K, OUT), jnp.float32),
        mesh=mesh,
        scratch_types=[
            pltpu.VMEM((_ROWS_PER_W // _CHUNK, _CHUNK), jnp.int32),
            pltpu.VMEM((_CHUNK, OUT), jnp.float32),
            pltpu.SemaphoreType.DMA,
        ],
    )
    return kern(table, idx2)


_RB = 128


def _reduce_body(g_ref, cent_ref, ws_ref, pmax_ref, pmin_ref, s_ref, sq_ref):
    k = pl.program_id(2)
    q = jnp.dot(cent_ref[0, 0], ws_ref[...],
                preferred_element_type=jnp.float32)
    t = g_ref[0, 0] - q
    rs = jnp.sum(t, axis=-1, keepdims=True)
    rsq = jnp.sum(t * t, axis=-1, keepdims=True)

    @pl.when(k == 0)
    def _init():
        pmax_ref[0, 0] = t
        pmin_ref[0, 0] = t
        s_ref[0, 0] = rs
        sq_ref[0, 0] = rsq

    @pl.when(k > 0)
    def _acc():
        pmax_ref[0, 0] = jnp.maximum(pmax_ref[0, 0], t)
        pmin_ref[0, 0] = jnp.minimum(pmin_ref[0, 0], t)
        s_ref[0, 0] = s_ref[0, 0] + rs
        sq_ref[0, 0] = sq_ref[0, 0] + rsq


def _reduce5(g4, cent4, ws_t):
    nrb = HALF // _RB
    grid = (B, nrb, K)
    return pl.pallas_call(
        _reduce_body,
        grid=grid,
        out_shape=(
            jax.ShapeDtypeStruct((B, nrb, _RB, OUT), jnp.float32),
            jax.ShapeDtypeStruct((B, nrb, _RB, OUT), jnp.float32),
            jax.ShapeDtypeStruct((B, nrb, _RB, 1), jnp.float32),
            jax.ShapeDtypeStruct((B, nrb, _RB, 1), jnp.float32),
        ),
        in_specs=[
            pl.BlockSpec((1, 1, _RB, OUT), lambda b, j, k: (k, b * nrb + j, 0, 0)),
            pl.BlockSpec((1, 1, _RB, SDIM), lambda b, j, k: (b, j, 0, 0)),
            pl.BlockSpec((SDIM, OUT), lambda b, j, k: (0, 0)),
        ],
        out_specs=(
            pl.BlockSpec((1, 1, _RB, OUT), lambda b, j, k: (b, j, 0, 0)),
            pl.BlockSpec((1, 1, _RB, OUT), lambda b, j, k: (b, j, 0, 0)),
            pl.BlockSpec((1, 1, _RB, 1), lambda b, j, k: (b, j, 0, 0)),
            pl.BlockSpec((1, 1, _RB, 1), lambda b, j, k: (b, j, 0, 0)),
        ),
    )(g4, cent4, ws_t)


def _final_body(pmax_ref, pmin_ref, s_ref, sq_ref, gam_ref, bet_ref, out_ref):
    cnt = jnp.float32(B * K * OUT)
    mean = jnp.sum(s_ref[...], axis=-1, keepdims=True) / cnt
    msq = jnp.sum(sq_ref[...], axis=-1, keepdims=True) / cnt
    var = msq - mean * mean
    scale = gam_ref[...] * lax.rsqrt(var + 1e-5)
    shift = bet_ref[...] - mean * scale
    val = jnp.where(scale >= 0, pmax_ref[...], pmin_ref[...]) * scale + shift
    out_ref[...] = jnp.maximum(val, 0.0)


def _finalize(pmax, pmin, s_t, sq_t, gam3, bet3):
    nrb = HALF // _RB
    grid = (nrb,)
    return pl.pallas_call(
        _final_body,
        grid=grid,
        out_shape=jax.ShapeDtypeStruct((B, nrb, _RB, OUT), jnp.float32),
        in_specs=[
            pl.BlockSpec((B, 1, _RB, OUT), lambda j: (0, j, 0, 0)),
            pl.BlockSpec((B, 1, _RB, OUT), lambda j: (0, j, 0, 0)),
            pl.BlockSpec((1, _RB, B), lambda j: (j, 0, 0)),
            pl.BlockSpec((1, _RB, B), lambda j: (j, 0, 0)),
            pl.BlockSpec((1, _RB, 1), lambda j: (j, 0, 0)),
            pl.BlockSpec((1, _RB, 1), lambda j: (j, 0, 0)),
        ],
        out_specs=pl.BlockSpec((B, 1, _RB, OUT), lambda j: (0, j, 0, 0)),
    )(pmax, pmin, s_t, sq_t, gam3, bet3)


def kernel(x, spatial, W, b, gamma, beta):
    sp_t = spatial.transpose(0, 2, 1)
    idx = _fps(sp_t)
    keep, cents = _knn(sp_t, idx.reshape(B, HALF, 1))

    wf_t = W[:, :FEAT].T
    ws_t = W[:, FEAT:].T
    z = _linear(x, spatial, wf_t, ws_t, b.reshape(1, OUT))

    keep_g = keep + (jnp.arange(B, dtype=jnp.int32) * N)[:, None, None]
    idx_perm = keep_g.transpose(2, 0, 1).reshape(-1)
    g = _sc_gather(z.reshape(B * N, OUT),
                   idx_perm.reshape(-1, _CHUNK))

    nrb = HALF // _RB
    g4 = g.reshape(K, B * nrb, _RB, OUT)
    pmax, pmin, s, sq = _reduce5(g4, cents.reshape(B, nrb, _RB, SDIM), ws_t)

    s_t = s.reshape(B, HALF).T.reshape(nrb, _RB, B)
    sq_t = sq.reshape(B, HALF).T.reshape(nrb, _RB, B)
    out = _finalize(pmax, pmin, s_t, sq_t,
                    gamma.reshape(nrb, _RB, 1), beta.reshape(nrb, _RB, 1))
    return out.reshape(B, HALF, OUT), cents


# speedup vs baseline: 7.0640x; 1.0997x over previous
import functools

import jax
import jax.numpy as jnp
from jax import lax
from jax.experimental import pallas as pl
from jax.experimental.pallas import tpu as pltpu
from jax.experimental.pallas import tpu_sc as plsc

B = 4
N = 2048
HALF = N // 2
FEAT = 256
SDIM = 5
OUT = 512
K = 16

_NC = 2
_NS = 16
_NW = _NC * _NS
_ROWS_PER_W = (B * HALF * K) // _NW
_CHUNK = 64


def _fps_body(spt_ref, out_ref):
    spt = spt_ref[...]
    iota = lax.broadcasted_iota(jnp.int32, (B, N), 1)
    iota_h = lax.broadcasted_iota(jnp.int32, (B, HALF), 1)
    onehot0 = (iota == 0).astype(jnp.float32)
    rem0 = 1.0 - onehot0
    dist0 = jnp.full((B, N), jnp.inf, dtype=jnp.float32)
    sel0 = jnp.zeros((B, HALF), jnp.int32)

    def body(i, carry):
        dist, rem, onehot, sel = carry
        sq = []
        for d in range(SDIM):
            row = spt[:, d, :]
            cd = jnp.sum(onehot * row, axis=-1, keepdims=True)
            diff = cd - row
            sq.append(diff * diff)
        d2 = ((sq[0] + sq[4]) + sq[2]) + (sq[1] + sq[3])
        dist = jnp.where(rem > 0, jnp.minimum(d2, dist), dist)
        masked = jnp.where(rem > 0, dist, -jnp.inf)
        m = jnp.max(masked, axis=-1, keepdims=True)
        g = jnp.min(jnp.where(masked == m, iota, jnp.int32(1 << 30)),
                    axis=-1, keepdims=True)
        sel = jnp.where(iota_h == i, g, sel)
        rank_f = jnp.sum(jnp.where(iota < g, rem, 0.0), axis=-1, keepdims=True)
        rank = rank_f.astype(jnp.int32)
        rem = jnp.where(iota == rank, 0.0, rem)
        onehot = (iota == g).astype(jnp.float32)
        return dist, rem, onehot, sel

    _, _, _, sel = lax.fori_loop(1, HALF, body, (dist0, rem0, onehot0, sel0),
                                 unroll=3)
    out_ref[...] = sel


def _fps(sp_t):
    return pl.pallas_call(
        _fps_body,
        out_shape=jax.ShapeDtypeStruct((B, HALF), jnp.int32),
        in_specs=[pl.BlockSpec((B, SDIM, N), lambda: (0, 0, 0))],
        out_specs=pl.BlockSpec((B, HALF), lambda: (0, 0)),
    )(sp_t)


_CB = 256
_NCB = HALF // _CB


def _knn_body(spt_ref, idx_ref, keep_ref, cent_ref):
    spt = spt_ref[0]
    cidx = idx_ref[0]
    iota = lax.broadcasted_iota(jnp.int32, (_CB, N), 1)
    onehot = (iota == cidx).astype(jnp.float32)

    sq = []
    cols = []
    for d in range(SDIM):
        row = spt[d:d + 1, :]
        cd = jnp.sum(onehot * row, axis=-1, keepdims=True)
        cols.append(cd)
        diff = cd - row
        sq.append(diff * diff)
    cent_ref[0] = jnp.concatenate(cols, axis=-1)
    acc = ((sq[0] + sq[4]) + sq[2]) + (sq[1] + sq[3])
    dist = jnp.sqrt(acc)

    idxs = []
    for _ in range(K + 1):
        m = jnp.min(dist, axis=-1, keepdims=True)
        j = jnp.min(jnp.where(dist == m, iota, jnp.int32(1 << 30)),
                    axis=-1, keepdims=True)
        idxs.append(j)
        dist = jnp.where(iota == j, jnp.inf, dist)

    cum = jnp.zeros_like(cidx, dtype=jnp.bool_)
    keep = []
    for t in range(K):
        cum = jnp.logical_or(cum, idxs[t] == cidx)
        keep.append(jnp.where(cum, idxs[t + 1], idxs[t]))
    keep_ref[0] = jnp.concatenate(keep, axis=-1)


def _knn(sp_t, idx3):
    grid = (B, _NCB)
    return pl.pallas_call(
        _knn_body,
        grid=grid,
        out_shape=(
            jax.ShapeDtypeStruct((B, HALF, K), jnp.int32),
            jax.ShapeDtypeStruct((B, HALF, SDIM), jnp.float32),
        ),
        in_specs=[
            pl.BlockSpec((1, SDIM, N), lambda b, j: (b, 0, 0)),
            pl.BlockSpec((1, _CB, 1), lambda b, j: (b, j, 0)),
        ],
        out_specs=(
            pl.BlockSpec((1, _CB, K), lambda b, j: (b, j, 0)),
            pl.BlockSpec((1, _CB, SDIM), lambda b, j: (b, j, 0)),
        ),
    )(sp_t, idx3)


_LB = 256


def _linear_body(x_ref, sp_ref, wf_ref, ws_ref, b_ref, z_ref):
    z = jnp.dot(x_ref[0], wf_ref[...], preferred_element_type=jnp.float32)
    z = z + jnp.dot(sp_ref[0], ws_ref[...], preferred_element_type=jnp.float32)
    z_ref[0] = z + b_ref[...]


def _linear(x, sp, wf_t, ws_t, bias2):
    grid = (B, N // _LB)
    return pl.pallas_call(
        _linear_body,
        grid=grid,
        out_shape=jax.ShapeDtypeStruct((B, N, OUT), jnp.float32),
        in_specs=[
            pl.BlockSpec((1, _LB, FEAT), lambda b, j: (b, j, 0)),
            pl.BlockSpec((1, _LB, SDIM), lambda b, j: (b, j, 0)),
            pl.BlockSpec((FEAT, OUT), lambda b, j: (0, 0)),
            pl.BlockSpec((SDIM, OUT), lambda b, j: (0, 0)),
            pl.BlockSpec((1, OUT), lambda b, j: (0, 0)),
        ],
        out_specs=pl.BlockSpec((1, _LB, OUT), lambda b, j: (b, j, 0)),
    )(x, sp, wf_t, ws_t, bias2)


def _sc_gather_body(table_hbm, idx_hbm, out_hbm, idx_v, rows_v, sem):
    wid = lax.axis_index("s") * _NC + lax.axis_index("c")
    nchunks = _ROWS_PER_W // _CHUNK
    pltpu.sync_copy(idx_hbm.at[pl.ds(wid * nchunks, nchunks)], idx_v)

    pltpu.async_copy(table_hbm.at[idx_v.at[0]], rows_v.at[0], sem)

    def chunk(j, _):
        pltpu.make_async_copy(table_hbm.at[idx_v.at[j]],
                              rows_v.at[j % 2], sem).wait()

        @pl.when(j + 1 < nchunks)
        def _next():
            pltpu.async_copy(table_hbm.at[idx_v.at[j + 1]],
                             rows_v.at[(j + 1) % 2], sem)

        pltpu.sync_copy(
            rows_v.at[j % 2],
            out_hbm.at[pl.ds(wid * _ROWS_PER_W + j * _CHUNK, _CHUNK)])
        return 0

    lax.fori_loop(0, nchunks, chunk, 0, unroll=False)


def _sc_gather(table, idx2):
    mesh = plsc.VectorSubcoreMesh(
        core_axis_name="c", subcore_axis_name="s",
        num_cores=_NC, num_subcores=_NS)
    kern = pl.kernel(
        _sc_gather_body,
        out_type=jax.ShapeDtypeStruct((B * HALF * K, OUT), jnp.float32),
        mesh=mesh,
        scratch_types=[
            pltpu.VMEM((_ROWS_PER_W // _CHUNK, _CHUNK), jnp.int32),
            pltpu.VMEM((2, _CHUNK, OUT), jnp.float32),
            pltpu.SemaphoreType.DMA,
        ],
    )
    return kern(table, idx2)


_RB = 128


def _reduce_body(g_ref, cent_ref, ws_ref, pmax_ref, pmin_ref, s_ref, sq_ref):
    k = pl.program_id(2)
    q = jnp.dot(cent_ref[0, 0], ws_ref[...],
                preferred_element_type=jnp.float32)
    t = g_ref[0, 0] - q
    rs = jnp.sum(t, axis=-1, keepdims=True)
    rsq = jnp.sum(t * t, axis=-1, keepdims=True)

    @pl.when(k == 0)
    def _init():
        pmax_ref[0, 0] = t
        pmin_ref[0, 0] = t
        s_ref[0, 0] = rs
        sq_ref[0, 0] = rsq

    @pl.when(k > 0)
    def _acc():
        pmax_ref[0, 0] = jnp.maximum(pmax_ref[0, 0], t)
        pmin_ref[0, 0] = jnp.minimum(pmin_ref[0, 0], t)
        s_ref[0, 0] = s_ref[0, 0] + rs
        sq_ref[0, 0] = sq_ref[0, 0] + rsq


def _reduce5(g4, cent4, ws_t):
    nrb = HALF // _RB
    grid = (B, nrb, K)
    return pl.pallas_call(
        _reduce_body,
        grid=grid,
        out_shape=(
            jax.ShapeDtypeStruct((B, nrb, _RB, OUT), jnp.float32),
            jax.ShapeDtypeStruct((B, nrb, _RB, OUT), jnp.float32),
            jax.ShapeDtypeStruct((B, nrb, _RB, 1), jnp.float32),
            jax.ShapeDtypeStruct((B, nrb, _RB, 1), jnp.float32),
        ),
        in_specs=[
            pl.BlockSpec((1, 1, _RB, OUT), lambda b, j, k: (k, b * nrb + j, 0, 0)),
            pl.BlockSpec((1, 1, _RB, SDIM), lambda b, j, k: (b, j, 0, 0)),
            pl.BlockSpec((SDIM, OUT), lambda b, j, k: (0, 0)),
        ],
        out_specs=(
            pl.BlockSpec((1, 1, _RB, OUT), lambda b, j, k: (b, j, 0, 0)),
            pl.BlockSpec((1, 1, _RB, OUT), lambda b, j, k: (b, j, 0, 0)),
            pl.BlockSpec((1, 1, _RB, 1), lambda b, j, k: (b, j, 0, 0)),
            pl.BlockSpec((1, 1, _RB, 1), lambda b, j, k: (b, j, 0, 0)),
        ),
    )(g4, cent4, ws_t)


def _final_body(pmax_ref, pmin_ref, s_ref, sq_ref, gam_ref, bet_ref, out_ref):
    cnt = jnp.float32(B * K * OUT)
    mean = jnp.sum(s_ref[...], axis=-1, keepdims=True) / cnt
    msq = jnp.sum(sq_ref[...], axis=-1, keepdims=True) / cnt
    var = msq - mean * mean
    scale = gam_ref[...] * lax.rsqrt(var + 1e-5)
    shift = bet_ref[...] - mean * scale
    val = jnp.where(scale >= 0, pmax_ref[...], pmin_ref[...]) * scale + shift
    out_ref[...] = jnp.maximum(val, 0.0)


def _finalize(pmax, pmin, s_t, sq_t, gam3, bet3):
    nrb = HALF // _RB
    grid = (nrb,)
    return pl.pallas_call(
        _final_body,
        grid=grid,
        out_shape=jax.ShapeDtypeStruct((B, nrb, _RB, OUT), jnp.float32),
        in_specs=[
            pl.BlockSpec((B, 1, _RB, OUT), lambda j: (0, j, 0, 0)),
            pl.BlockSpec((B, 1, _RB, OUT), lambda j: (0, j, 0, 0)),
            pl.BlockSpec((1, _RB, B), lambda j: (j, 0, 0)),
            pl.BlockSpec((1, _RB, B), lambda j: (j, 0, 0)),
            pl.BlockSpec((1, _RB, 1), lambda j: (j, 0, 0)),
            pl.BlockSpec((1, _RB, 1), lambda j: (j, 0, 0)),
        ],
        out_specs=pl.BlockSpec((B, 1, _RB, OUT), lambda j: (0, j, 0, 0)),
    )(pmax, pmin, s_t, sq_t, gam3, bet3)


def kernel(x, spatial, W, b, gamma, beta):
    sp_t = spatial.transpose(0, 2, 1)
    idx = _fps(sp_t)
    keep, cents = _knn(sp_t, idx.reshape(B, HALF, 1))

    wf_t = W[:, :FEAT].T
    ws_t = W[:, FEAT:].T
    z = _linear(x, spatial, wf_t, ws_t, b.reshape(1, OUT))

    keep_g = keep + (jnp.arange(B, dtype=jnp.int32) * N)[:, None, None]
    idx_perm = keep_g.transpose(2, 0, 1).reshape(-1)
    g = _sc_gather(z.reshape(B * N, OUT),
                   idx_perm.reshape(-1, _CHUNK))

    nrb = HALF // _RB
    g4 = g.reshape(K, B * nrb, _RB, OUT)
    pmax, pmin, s, sq = _reduce5(g4, cents.reshape(B, nrb, _RB, SDIM), ws_t)

    s_t = s.reshape(B, HALF).T.reshape(nrb, _RB, B)
    sq_t = sq.reshape(B, HALF).T.reshape(nrb, _RB, B)
    out = _finalize(pmax, pmin, s_t, sq_t,
                    gamma.reshape(nrb, _RB, 1), beta.reshape(nrb, _RB, 1))
    return out.reshape(B, HALF, OUT), cents
